# baseline (speedup 1.0000x reference)
import jax
import jax.numpy as jnp
from jax.experimental import pallas as pl
from jax.experimental.pallas import tpu as pltpu

_B, _D, _S = 512, 2048, 8192
_L, _C = 64, 6
_R = _L * _C
_TS = 2048
_NT = _S // _TS
_POS = 1e30
_LOG2E = 1.4426950408889634
_LN2 = 0.6931471805599453


def _body(x_ref, seg_ref, f_ref, out_ref, cx_ref, d_ref, s_ref, emn_ref,
          cnt_ref):
    t = pl.program_id(0)

    @pl.when(t == 0)
    def _init():
        x = x_ref[...]
        xn = (x / jnp.sqrt(jnp.sum(x * x, axis=1, keepdims=True))).astype(
            jnp.bfloat16)
        rows = jax.lax.broadcasted_iota(jnp.int32, (_R, _B), 0)
        onehot = (seg_ref[...] == rows).astype(jnp.bfloat16)
        cnt = jnp.sum(onehot.astype(jnp.float32), axis=1, keepdims=True)
        cnt_ref[...] = cnt
        cx = jnp.dot(onehot, xn, preferred_element_type=jnp.float32)
        cx_ref[...] = (cx * ((20.0 * _LOG2E) / jnp.maximum(cnt, 1.0))
                       ).astype(jnp.bfloat16)
        s_ref[...] = jnp.zeros((_R, 1), jnp.float32)
        emn_ref[...] = jnp.full((_R, 1), _POS, jnp.float32)

    _CK = 512
    for k in range(_TS // _CK):
        p = jax.lax.dot_general(
            cx_ref[...],
            f_ref[pl.ds(k * _CK, _CK), :].astype(jnp.bfloat16),
            (((1,), (1,)), ((), ())), preferred_element_type=jnp.float32)
        e = jnp.exp2(p)
        s_ref[...] = s_ref[...] + jnp.sum(e, axis=1, keepdims=True)
        emn_ref[...] = jnp.minimum(
            emn_ref[...], jnp.exp2(jnp.min(p, axis=1, keepdims=True)))

        if k == 0:
            @pl.when(t == 0)
            def _diag():
                cols = jax.lax.broadcasted_iota(jnp.int32, (_R, _L), 1)
                lids = jax.lax.broadcasted_iota(jnp.int32, (_R, _L), 0) % _L
                d_ref[...] = jnp.sum(
                    jnp.where(cols == lids, p[:, 0:_L], 0.0),
                    axis=1, keepdims=True) * _LN2

    @pl.when(t == _NT - 1)
    def _finish():
        valid = cnt_ref[...] > 0.0
        dv = jnp.where(valid, d_ref[...], _POS)
        w = jnp.where(valid, s_ref[...] - emn_ref[...], 0.0)
        vf = valid.astype(jnp.float32)
        pos = dv[0:_L, :]
        wsum = w[0:_L, :]
        nv = vf[0:_L, :]
        for c in range(1, _C):
            pos = jnp.minimum(pos, dv[c * _L:(c + 1) * _L, :])
            wsum = wsum + w[c * _L:(c + 1) * _L, :]
            nv = nv + vf[c * _L:(c + 1) * _L, :]
        label_valid = nv > 0.0
        pos = jnp.where(label_valid, pos, 0.0)
        logp = pos - jnp.log(jnp.exp(pos) + wsum)
        num = jnp.sum(jnp.where(label_valid, -logp, 0.0), axis=(0, 1),
                      keepdims=True)
        den = jnp.sum(label_valid.astype(jnp.float32), axis=(0, 1),
                      keepdims=True)
        out_ref[...] = num / den


def _impl(inputs, targets, camids, features):
    seg = (camids * _L + targets).astype(jnp.int32).reshape(1, _B)
    loss = pl.pallas_call(
        _body,
        grid=(_NT,),
        in_specs=[
            pl.BlockSpec((_B, _D), lambda t: (0, 0)),
            pl.BlockSpec((1, _B), lambda t: (0, 0)),
            pl.BlockSpec((_TS, _D), lambda t: (t, 0)),
        ],
        out_specs=pl.BlockSpec((1, 1), lambda t: (0, 0)),
        out_shape=jax.ShapeDtypeStruct((1, 1), jnp.float32),
        scratch_shapes=[
            pltpu.VMEM((_R, _D), jnp.bfloat16),
            pltpu.VMEM((_R, 1), jnp.float32),
            pltpu.VMEM((_R, 1), jnp.float32),
            pltpu.VMEM((_R, 1), jnp.float32),
            pltpu.VMEM((_R, 1), jnp.float32),
        ],
    )(inputs, seg, features)
    return loss[0, 0]


def kernel(inputs, targets, camids, isClusterC, features):
    loss = _impl(inputs, targets, camids, features)
    return loss * jnp.asarray(isClusterC).astype(loss.dtype)


# speedup vs baseline: 1.0160x; 1.0160x over previous
import jax
import jax.numpy as jnp
from jax.experimental import pallas as pl
from jax.experimental.pallas import tpu as pltpu

_B, _D, _S = 512, 2048, 8192
_L, _C = 64, 6
_R = _L * _C
_TS = 1024
_NT = _S // _TS
_POS = 1e30
_LOG2E = 1.4426950408889634
_LN2 = 0.6931471805599453


def _body(x_ref, seg_ref, f_ref, out_ref, cx_ref, d_ref, s_ref, emn_ref,
          cnt_ref):
    t = pl.program_id(0)

    @pl.when(t == 0)
    def _init():
        x = x_ref[...]
        xn = (x / jnp.sqrt(jnp.sum(x * x, axis=1, keepdims=True))).astype(
            jnp.bfloat16)
        rows = jax.lax.broadcasted_iota(jnp.int32, (_R, _B), 0)
        onehot = (seg_ref[...] == rows).astype(jnp.bfloat16)
        cnt = jnp.sum(onehot.astype(jnp.float32), axis=1, keepdims=True)
        cnt_ref[...] = cnt
        cx = jnp.dot(onehot, xn, preferred_element_type=jnp.float32)
        cx_ref[...] = (cx * ((20.0 * _LOG2E) / jnp.maximum(cnt, 1.0))
                       ).astype(jnp.bfloat16)
        s_ref[...] = jnp.zeros((_R, 1), jnp.float32)
        emn_ref[...] = jnp.full((_R, 1), _POS, jnp.float32)

    p = jax.lax.dot_general(cx_ref[...], f_ref[...].astype(jnp.bfloat16),
                            (((1,), (1,)), ((), ())),
                            preferred_element_type=jnp.float32)
    e = jnp.exp2(p)
    s_ref[...] = s_ref[...] + jnp.sum(e, axis=1, keepdims=True)
    emn_ref[...] = jnp.minimum(
        emn_ref[...], jnp.exp2(jnp.min(p, axis=1, keepdims=True)))

    @pl.when(t == 0)
    def _diag():
        cols = jax.lax.broadcasted_iota(jnp.int32, (_R, _L), 1)
        lids = jax.lax.broadcasted_iota(jnp.int32, (_R, _L), 0) % _L
        d_ref[...] = jnp.sum(jnp.where(cols == lids, p[:, 0:_L], 0.0),
                             axis=1, keepdims=True) * _LN2

    @pl.when(t == _NT - 1)
    def _finish():
        valid = cnt_ref[...] > 0.0
        dv = jnp.where(valid, d_ref[...], _POS)
        w = jnp.where(valid, s_ref[...] - emn_ref[...], 0.0)
        vf = valid.astype(jnp.float32)
        pos = dv[0:_L, :]
        wsum = w[0:_L, :]
        nv = vf[0:_L, :]
        for c in range(1, _C):
            pos = jnp.minimum(pos, dv[c * _L:(c + 1) * _L, :])
            wsum = wsum + w[c * _L:(c + 1) * _L, :]
            nv = nv + vf[c * _L:(c + 1) * _L, :]
        label_valid = nv > 0.0
        pos = jnp.where(label_valid, pos, 0.0)
        logp = pos - jnp.log(jnp.exp(pos) + wsum)
        num = jnp.sum(jnp.where(label_valid, -logp, 0.0), axis=(0, 1),
                      keepdims=True)
        den = jnp.sum(label_valid.astype(jnp.float32), axis=(0, 1),
                      keepdims=True)
        out_ref[...] = num / den


def _impl(inputs, targets, camids, features):
    seg = (camids * _L + targets).astype(jnp.int32).reshape(1, _B)
    loss = pl.pallas_call(
        _body,
        grid=(_NT,),
        in_specs=[
            pl.BlockSpec((_B, _D), lambda t: (0, 0)),
            pl.BlockSpec((1, _B), lambda t: (0, 0)),
            pl.BlockSpec((_TS, _D), lambda t: (t, 0)),
        ],
        out_specs=pl.BlockSpec((1, 1), lambda t: (0, 0)),
        out_shape=jax.ShapeDtypeStruct((1, 1), jnp.float32),
        scratch_shapes=[
            pltpu.VMEM((_R, _D), jnp.bfloat16),
            pltpu.VMEM((_R, 1), jnp.float32),
            pltpu.VMEM((_R, 1), jnp.float32),
            pltpu.VMEM((_R, 1), jnp.float32),
            pltpu.VMEM((_R, 1), jnp.float32),
        ],
    )(inputs, seg, features)
    return loss[0, 0]


def kernel(inputs, targets, camids, isClusterC, features):
    loss = _impl(inputs, targets, camids, features)
    return loss * jnp.asarray(isClusterC).astype(loss.dtype)


# speedup vs baseline: 1.0323x; 1.0160x over previous
import jax
import jax.numpy as jnp
from jax.experimental import pallas as pl
from jax.experimental.pallas import tpu as pltpu

_B, _D, _S = 512, 2048, 8192
_L, _C = 64, 6
_R = _L * _C
_TS = 2048
_NT = _S // _TS
_POS = 1e30
_LOG2E = 1.4426950408889634
_LN2 = 0.6931471805599453


def _body(x_ref, seg_ref, f_ref, out_ref, cx_ref, d_ref, s_ref, emn_ref,
          cnt_ref):
    t = pl.program_id(0)

    @pl.when(t == 0)
    def _init():
        x = x_ref[...]
        xn = (x / jnp.sqrt(jnp.sum(x * x, axis=1, keepdims=True))).astype(
            jnp.bfloat16)
        rows = jax.lax.broadcasted_iota(jnp.int32, (_R, _B), 0)
        onehot = (seg_ref[...] == rows).astype(jnp.bfloat16)
        cnt = jnp.sum(onehot.astype(jnp.float32), axis=1, keepdims=True)
        cnt_ref[...] = cnt
        cx = jnp.dot(onehot, xn, preferred_element_type=jnp.float32)
        cx_ref[...] = (cx * ((20.0 * _LOG2E) / jnp.maximum(cnt, 1.0))
                       ).astype(jnp.bfloat16)
        s_ref[...] = jnp.zeros((_R, 1), jnp.float32)
        emn_ref[...] = jnp.full((_R, 1), _POS, jnp.float32)

    p = jax.lax.dot_general(cx_ref[...], f_ref[...].astype(jnp.bfloat16),
                            (((1,), (1,)), ((), ())),
                            preferred_element_type=jnp.float32)
    e = jnp.exp2(p)
    s_ref[...] = s_ref[...] + jnp.sum(e, axis=1, keepdims=True)
    emn_ref[...] = jnp.minimum(
        emn_ref[...], jnp.exp2(jnp.min(p, axis=1, keepdims=True)))

    @pl.when(t == 0)
    def _diag():
        cols = jax.lax.broadcasted_iota(jnp.int32, (_R, _L), 1)
        lids = jax.lax.broadcasted_iota(jnp.int32, (_R, _L), 0) % _L
        d_ref[...] = jnp.sum(jnp.where(cols == lids, p[:, 0:_L], 0.0),
                             axis=1, keepdims=True) * _LN2

    @pl.when(t == _NT - 1)
    def _finish():
        valid = cnt_ref[...] > 0.0
        dv = jnp.where(valid, d_ref[...], _POS)
        w = jnp.where(valid, s_ref[...] - emn_ref[...], 0.0)
        vf = valid.astype(jnp.float32)
        pos = dv[0:_L, :]
        wsum = w[0:_L, :]
        nv = vf[0:_L, :]
        for c in range(1, _C):
            pos = jnp.minimum(pos, dv[c * _L:(c + 1) * _L, :])
            wsum = wsum + w[c * _L:(c + 1) * _L, :]
            nv = nv + vf[c * _L:(c + 1) * _L, :]
        label_valid = nv > 0.0
        pos = jnp.where(label_valid, pos, 0.0)
        logp = pos - jnp.log(jnp.exp(pos) + wsum)
        num = jnp.sum(jnp.where(label_valid, -logp, 0.0), axis=(0, 1),
                      keepdims=True)
        den = jnp.sum(label_valid.astype(jnp.float32), axis=(0, 1),
                      keepdims=True)
        out_ref[...] = num / den


def _impl(inputs, targets, camids, features):
    seg = (camids * _L + targets).astype(jnp.int32).reshape(1, _B)
    loss = pl.pallas_call(
        _body,
        grid=(_NT,),
        in_specs=[
            pl.BlockSpec((_B, _D), lambda t: (0, 0)),
            pl.BlockSpec((1, _B), lambda t: (0, 0)),
            pl.BlockSpec((_TS, _D), lambda t: (t, 0)),
        ],
        out_specs=pl.BlockSpec((1, 1), lambda t: (0, 0)),
        out_shape=jax.ShapeDtypeStruct((1, 1), jnp.float32),
        scratch_shapes=[
            pltpu.VMEM((_R, _D), jnp.bfloat16),
            pltpu.VMEM((_R, 1), jnp.float32),
            pltpu.VMEM((_R, 1), jnp.float32),
            pltpu.VMEM((_R, 1), jnp.float32),
            pltpu.VMEM((_R, 1), jnp.float32),
        ],
    )(inputs, seg, features)
    return loss[0, 0]


def kernel(inputs, targets, camids, isClusterC, features):
    loss = _impl(inputs, targets, camids, features)
    return loss * jnp.asarray(isClusterC).astype(loss.dtype)
